# baseline (speedup 1.0000x reference)
import jax
import jax.numpy as jnp
from jax.experimental import pallas as pl
from jax.experimental.pallas import tpu as pltpu

_NCHUNK = 8


def _body(msg_ref, ent_ref, out_ref, ment_ref, sent_ref, mnn_ref, snn_ref,
          sems):
    B = msg_ref.shape[0]
    chunk = B // _NCHUNK
    copies = []
    for c in range(_NCHUNK):
        cp = pltpu.make_async_copy(
            msg_ref.at[pl.ds(c * chunk, chunk), :],
            out_ref.at[pl.ds(c * chunk, chunk), :],
            sems.at[c],
        )
        cp.start()
        copies.append(cp)
    e = ent_ref[...]
    s = jnp.sum(e, axis=1, keepdims=True)
    ment_ref[...] = s
    sent_ref[...] = e
    mnn_ref[...] = s
    snn_ref[...] = e
    for cp in copies:
        cp.wait()


def kernel(messages, apply_noise, entropy):
    B, L, V = messages.shape
    msg2d = messages.reshape(B, L * V)
    out2d, ment, sent, mnn, snn = pl.pallas_call(
        _body,
        in_specs=[
            pl.BlockSpec(memory_space=pl.ANY),
            pl.BlockSpec((B, L), lambda: (0, 0)),
        ],
        out_specs=[
            pl.BlockSpec(memory_space=pl.ANY),
            pl.BlockSpec((B, 1), lambda: (0, 0)),
            pl.BlockSpec((B, L), lambda: (0, 0)),
            pl.BlockSpec((B, 1), lambda: (0, 0)),
            pl.BlockSpec((B, L), lambda: (0, 0)),
        ],
        out_shape=[
            jax.ShapeDtypeStruct((B, L * V), messages.dtype),
            jax.ShapeDtypeStruct((B, 1), entropy.dtype),
            jax.ShapeDtypeStruct((B, L), entropy.dtype),
            jax.ShapeDtypeStruct((B, 1), entropy.dtype),
            jax.ShapeDtypeStruct((B, L), entropy.dtype),
        ],
        scratch_shapes=[pltpu.SemaphoreType.DMA((_NCHUNK,))],
    )(msg2d, entropy)
    return (
        out2d.reshape(B, L, V),
        ment.reshape(B),
        sent,
        mnn.reshape(B),
        snn,
    )


# speedup vs baseline: 14.4596x; 14.4596x over previous
import jax
import jax.numpy as jnp
from jax.experimental import pallas as pl

_TB = 128


def _body(msg_ref, ent_ref, out_ref, ment_ref, sent_ref, mnn_ref, snn_ref):
    out_ref[...] = msg_ref[...]
    e = ent_ref[...]
    s = jnp.sum(e, axis=1, keepdims=True)
    ment_ref[...] = s
    sent_ref[...] = e
    mnn_ref[...] = s
    snn_ref[...] = e


def kernel(messages, apply_noise, entropy):
    B, L, V = messages.shape
    msg2d = messages.reshape(B, L * V)
    grid = (B // _TB,)
    out2d, ment, sent, mnn, snn = pl.pallas_call(
        _body,
        grid=grid,
        in_specs=[
            pl.BlockSpec((_TB, L * V), lambda i: (i, 0)),
            pl.BlockSpec((_TB, L), lambda i: (i, 0)),
        ],
        out_specs=[
            pl.BlockSpec((_TB, L * V), lambda i: (i, 0)),
            pl.BlockSpec((_TB, 1), lambda i: (i, 0)),
            pl.BlockSpec((_TB, L), lambda i: (i, 0)),
            pl.BlockSpec((_TB, 1), lambda i: (i, 0)),
            pl.BlockSpec((_TB, L), lambda i: (i, 0)),
        ],
        out_shape=[
            jax.ShapeDtypeStruct((B, L * V), messages.dtype),
            jax.ShapeDtypeStruct((B, 1), entropy.dtype),
            jax.ShapeDtypeStruct((B, L), entropy.dtype),
            jax.ShapeDtypeStruct((B, 1), entropy.dtype),
            jax.ShapeDtypeStruct((B, L), entropy.dtype),
        ],
    )(msg2d, entropy)
    return (
        out2d.reshape(B, L, V),
        ment.reshape(B),
        sent,
        mnn.reshape(B),
        snn,
    )


# speedup vs baseline: 43.5437x; 3.0114x over previous
import jax
import jax.numpy as jnp
from jax.experimental import pallas as pl

_TB = 256


def _body(msg_ref, ent_ref, out_ref, ment_ref, sent_ref, mnn_ref, snn_ref):
    out_ref[...] = msg_ref[...]
    e = ent_ref[...]
    s = jnp.sum(e, axis=1, keepdims=True)
    ment_ref[...] = s
    sent_ref[...] = e
    mnn_ref[...] = s
    snn_ref[...] = e


def kernel(messages, apply_noise, entropy):
    B, L, V = messages.shape
    grid = (B // _TB,)
    out, ment, sent, mnn, snn = pl.pallas_call(
        _body,
        grid=grid,
        in_specs=[
            pl.BlockSpec((_TB, L, V), lambda i: (i, 0, 0)),
            pl.BlockSpec((_TB, L), lambda i: (i, 0)),
        ],
        out_specs=[
            pl.BlockSpec((_TB, L, V), lambda i: (i, 0, 0)),
            pl.BlockSpec((_TB, 1), lambda i: (i, 0)),
            pl.BlockSpec((_TB, L), lambda i: (i, 0)),
            pl.BlockSpec((_TB, 1), lambda i: (i, 0)),
            pl.BlockSpec((_TB, L), lambda i: (i, 0)),
        ],
        out_shape=[
            jax.ShapeDtypeStruct((B, L, V), messages.dtype),
            jax.ShapeDtypeStruct((B, 1), entropy.dtype),
            jax.ShapeDtypeStruct((B, L), entropy.dtype),
            jax.ShapeDtypeStruct((B, 1), entropy.dtype),
            jax.ShapeDtypeStruct((B, L), entropy.dtype),
        ],
    )(messages, entropy)
    return (
        out,
        ment.reshape(B),
        sent,
        mnn.reshape(B),
        snn,
    )
